# baseline (speedup 1.0000x reference)
import functools

import jax
import jax.numpy as jnp
from jax import lax
from jax.experimental import pallas as pl
from jax.experimental.pallas import tpu as pltpu
from jax.experimental.pallas import tpu_sc as plsc

N_EMB = 1024
DIM = 64
ROWS = 8 * 576
BLOCK = 576
N_BLOCKS = ROWS // BLOCK

N_WORKERS = 32
_B_PER_W = ROWS // N_WORKERS
_CH = _B_PER_W // 2
LOSS_SCALE = 0.5 / (ROWS * DIM)


def _tc_body(x_ref, wt_ref, idx_ref, loss_ref):
    x = x_ref[...]
    wt = wt_ref[...]

    num = jnp.dot(x, wt, preferred_element_type=jnp.float32)
    xsq = jnp.sum(x * x, axis=1, keepdims=True)
    wnsq = jnp.sum(wt * wt, axis=0, keepdims=True)
    x_norm = jnp.sqrt(xsq)
    w_norm = jnp.sqrt(wnsq)
    denom = jnp.maximum(x_norm * w_norm, 1e-8)
    dist = 1.0 - num / denom

    m = jnp.min(dist, axis=1, keepdims=True)
    iota = lax.broadcasted_iota(jnp.int32, (BLOCK, N_EMB), 1)
    idx = jnp.min(jnp.where(dist == m, iota, N_EMB), axis=1, keepdims=True)
    idx_ref[...] = idx

    sel = iota == idx
    num_sel = jnp.sum(jnp.where(sel, num, 0.0), axis=1)
    wnsq_sel = jnp.sum(jnp.where(sel, wnsq, 0.0), axis=1)
    block_loss = jnp.sum(xsq[:, 0] - 2.0 * num_sel + wnsq_sel)

    i = pl.program_id(0)

    @pl.when(i == 0)
    def _():
        loss_ref[0, 0] = 0.0

    loss_ref[0, 0] += block_loss * LOSS_SCALE


def _tc_call(flat, wt):
    return pl.pallas_call(
        _tc_body,
        grid=(N_BLOCKS,),
        in_specs=[
            pl.BlockSpec((BLOCK, DIM), lambda i: (i, 0)),
            pl.BlockSpec((DIM, N_EMB), lambda i: (0, 0)),
        ],
        out_specs=[
            pl.BlockSpec((BLOCK, 1), lambda i: (i, 0)),
            pl.BlockSpec((1, 1), lambda i: (0, 0), memory_space=pltpu.SMEM),
        ],
        out_shape=[
            jax.ShapeDtypeStruct((ROWS, 1), jnp.int32),
            jax.ShapeDtypeStruct((1, 1), jnp.float32),
        ],
    )(flat, wt)


def _sc_body(table_hbm, idx_hbm, out_hbm, idx_v, table_v, rows_v, sem):
    wid = lax.axis_index("s") * 2 + lax.axis_index("c")
    base = wid * _B_PER_W
    cpt = pltpu.async_copy(table_hbm, table_v, sem)
    pltpu.sync_copy(idx_hbm.at[pl.ds(base, _B_PER_W)], idx_v)
    cpt.wait()

    lanes = lax.iota(jnp.int32, 16)
    zeros = jnp.zeros((16,), jnp.int32)

    def block(b, carry):
        rows16 = lanes + 16 * b
        ridx = plsc.load_gather(idx_v, [rows16, zeros])
        prow = ridx >> 1
        pbase = (ridx & 1) * DIM
        for c in range(DIM):
            vals = plsc.load_gather(table_v, [prow, pbase + c])
            plsc.store_scatter(rows_v, [rows16, zeros + c], vals)
        return carry

    lax.fori_loop(0, _B_PER_W // 16, block, 0)
    pltpu.sync_copy(rows_v, out_hbm.at[pl.ds(base, _B_PER_W)])


@functools.cache
def _sc_gather():
    return pl.kernel(
        _sc_body,
        out_type=jax.ShapeDtypeStruct((ROWS, DIM), jnp.float32),
        mesh=plsc.VectorSubcoreMesh(core_axis_name="c", subcore_axis_name="s"),
        scratch_types=[
            pltpu.VMEM((_B_PER_W, 1), jnp.int32),
            pltpu.VMEM((N_EMB // 2, 2 * DIM), jnp.float32),
            pltpu.VMEM((_B_PER_W, DIM), jnp.float32),
            pltpu.SemaphoreType.DMA,
        ],
        compiler_params=pltpu.CompilerParams(needs_layout_passes=False),
    )


def _tc_full_body(x_ref, wt_ref, w_ref, q_ref, idx_ref, loss_ref):
    i = pl.program_id(0)
    x = x_ref[...]
    wt = wt_ref[...]

    num = jnp.dot(x, wt, preferred_element_type=jnp.float32)
    x_norm = jnp.sqrt(jnp.sum(x * x, axis=1, keepdims=True))
    w_norm = jnp.sqrt(jnp.sum(wt * wt, axis=0, keepdims=True))
    denom = jnp.maximum(x_norm * w_norm, 1e-8)
    dist = 1.0 - num / denom

    m = jnp.min(dist, axis=1, keepdims=True)
    iota = lax.broadcasted_iota(jnp.int32, (BLOCK, N_EMB), 1)
    idx = jnp.min(jnp.where(dist == m, iota, N_EMB), axis=1, keepdims=True)
    idx_ref[...] = idx

    onehot = (iota == idx).astype(jnp.float32)
    q = jnp.dot(onehot, w_ref[...], preferred_element_type=jnp.float32)
    q_ref[...] = q

    d = q - x
    block_loss = jnp.sum(d * d)

    @pl.when(i == 0)
    def _():
        loss_ref[0, 0] = 0.0

    loss_ref[0, 0] += block_loss * LOSS_SCALE


def _tc_full_call(flat, wt, w):
    return pl.pallas_call(
        _tc_full_body,
        grid=(N_BLOCKS,),
        in_specs=[
            pl.BlockSpec((BLOCK, DIM), lambda i: (i, 0)),
            pl.BlockSpec((DIM, N_EMB), lambda i: (0, 0)),
            pl.BlockSpec((N_EMB, DIM), lambda i: (0, 0)),
        ],
        out_specs=[
            pl.BlockSpec((BLOCK, DIM), lambda i: (i, 0)),
            pl.BlockSpec((BLOCK, 1), lambda i: (i, 0)),
            pl.BlockSpec((1, 1), lambda i: (0, 0), memory_space=pltpu.SMEM),
        ],
        out_shape=[
            jax.ShapeDtypeStruct((ROWS, DIM), jnp.float32),
            jax.ShapeDtypeStruct((ROWS, 1), jnp.int32),
            jax.ShapeDtypeStruct((1, 1), jnp.float32),
        ],
    )(flat, wt, w)


def kernel(inputs, weight):
    flat = inputs.reshape(ROWS, DIM)
    idx, loss = _tc_call(flat, weight.T)
    quantized = _sc_gather()(weight.reshape(N_EMB // 2, 2 * DIM), idx)
    return (quantized.reshape(inputs.shape), loss[0, 0], idx)


# speedup vs baseline: 1.1005x; 1.1005x over previous
import functools

import jax
import jax.numpy as jnp
from jax import lax
from jax.experimental import pallas as pl
from jax.experimental.pallas import tpu as pltpu
from jax.experimental.pallas import tpu_sc as plsc

N_EMB = 1024
DIM = 64
ROWS = 8 * 576
BLOCK = 576
N_BLOCKS = ROWS // BLOCK

N_WORKERS = 32
_B_PER_W = ROWS // N_WORKERS
_CH = _B_PER_W // 2
LOSS_SCALE = 0.5 / (ROWS * DIM)


def _tc_body(x_ref, wt_ref, idx_ref, loss_ref):
    x = x_ref[...]
    wt = wt_ref[...]

    num = jnp.dot(x, wt, preferred_element_type=jnp.float32)
    xsq = jnp.sum(x * x, axis=1, keepdims=True)
    wnsq = jnp.sum(wt * wt, axis=0, keepdims=True)
    x_norm = jnp.sqrt(xsq)
    w_norm = jnp.sqrt(wnsq)
    denom = jnp.maximum(x_norm * w_norm, 1e-8)
    dist = 1.0 - num / denom

    m = jnp.min(dist, axis=1, keepdims=True)
    iota = lax.broadcasted_iota(jnp.int32, (BLOCK, N_EMB), 1)
    idx = jnp.min(jnp.where(dist == m, iota, N_EMB), axis=1, keepdims=True)
    idx_ref[...] = idx

    sel = iota == idx
    num_sel = jnp.sum(jnp.where(sel, num, 0.0), axis=1)
    wnsq_sel = jnp.sum(jnp.where(sel, wnsq, 0.0), axis=1)
    block_loss = jnp.sum(xsq[:, 0] - 2.0 * num_sel + wnsq_sel)

    i = pl.program_id(0)

    @pl.when(i == 0)
    def _():
        loss_ref[0, 0] = 0.0

    loss_ref[0, 0] += block_loss * LOSS_SCALE


def _tc_call(flat, wt):
    return pl.pallas_call(
        _tc_body,
        grid=(N_BLOCKS,),
        in_specs=[
            pl.BlockSpec((BLOCK, DIM), lambda i: (i, 0)),
            pl.BlockSpec((DIM, N_EMB), lambda i: (0, 0)),
        ],
        out_specs=[
            pl.BlockSpec((BLOCK, 1), lambda i: (i, 0)),
            pl.BlockSpec((1, 1), lambda i: (0, 0), memory_space=pltpu.SMEM),
        ],
        out_shape=[
            jax.ShapeDtypeStruct((ROWS, 1), jnp.int32),
            jax.ShapeDtypeStruct((1, 1), jnp.float32),
        ],
    )(flat, wt)


def _sc_body(table_hbm, idx_hbm, out_hbm, idx_v, pidx_v, rows2_v, rows_v,
             sem):
    wid = lax.axis_index("s") * 2 + lax.axis_index("c")
    base = wid * _B_PER_W
    pltpu.sync_copy(idx_hbm.at[pl.ds(base, _B_PER_W)], idx_v)

    lanes = lax.iota(jnp.int32, 16)
    zeros = jnp.zeros((16,), jnp.int32)

    def pack(b, carry):
        rows16 = lanes + 16 * b
        ridx = plsc.load_gather(idx_v, [rows16, zeros])
        plsc.store_scatter(pidx_v, [rows16 // _CH, rows16 % _CH], ridx >> 1)
        return carry

    lax.fori_loop(0, _B_PER_W // 16, pack, 0)
    cp0 = pltpu.async_copy(table_hbm.at[pidx_v.at[0]],
                           rows2_v.at[pl.ds(0, _CH)], sem)
    cp1 = pltpu.async_copy(table_hbm.at[pidx_v.at[1]],
                           rows2_v.at[pl.ds(_CH, _CH)], sem)
    cp0.wait()
    cp1.wait()

    def compact(b, carry):
        rows16 = lanes + 16 * b
        ridx = plsc.load_gather(idx_v, [rows16, zeros])
        off = (ridx & 1) * DIM
        for c in range(DIM):
            vals = plsc.load_gather(rows2_v, [rows16, off + c])
            plsc.store_scatter(rows_v, [rows16, zeros + c], vals)
        return carry

    lax.fori_loop(0, _B_PER_W // 16, compact, 0)
    pltpu.sync_copy(rows_v, out_hbm.at[pl.ds(base, _B_PER_W)])


@functools.cache
def _sc_gather():
    return pl.kernel(
        _sc_body,
        out_type=jax.ShapeDtypeStruct((ROWS, DIM), jnp.float32),
        mesh=plsc.VectorSubcoreMesh(core_axis_name="c", subcore_axis_name="s"),
        scratch_types=[
            pltpu.VMEM((_B_PER_W, 1), jnp.int32),
            pltpu.VMEM((2, _CH), jnp.int32),
            pltpu.VMEM((_B_PER_W, 2 * DIM), jnp.float32),
            pltpu.VMEM((_B_PER_W, DIM), jnp.float32),
            pltpu.SemaphoreType.DMA,
        ],
        compiler_params=pltpu.CompilerParams(needs_layout_passes=False),
    )


def _tc_full_body(x_ref, wt_ref, w_ref, q_ref, idx_ref, loss_ref):
    i = pl.program_id(0)
    x = x_ref[...]
    wt = wt_ref[...]

    num = jnp.dot(x, wt, preferred_element_type=jnp.float32)
    x_norm = jnp.sqrt(jnp.sum(x * x, axis=1, keepdims=True))
    w_norm = jnp.sqrt(jnp.sum(wt * wt, axis=0, keepdims=True))
    denom = jnp.maximum(x_norm * w_norm, 1e-8)
    dist = 1.0 - num / denom

    m = jnp.min(dist, axis=1, keepdims=True)
    iota = lax.broadcasted_iota(jnp.int32, (BLOCK, N_EMB), 1)
    idx = jnp.min(jnp.where(dist == m, iota, N_EMB), axis=1, keepdims=True)
    idx_ref[...] = idx

    onehot = (iota == idx).astype(jnp.float32)
    q = jnp.dot(onehot, w_ref[...], preferred_element_type=jnp.float32)
    q_ref[...] = q

    d = q - x
    block_loss = jnp.sum(d * d)

    @pl.when(i == 0)
    def _():
        loss_ref[0, 0] = 0.0

    loss_ref[0, 0] += block_loss * LOSS_SCALE


def _tc_full_call(flat, wt, w):
    return pl.pallas_call(
        _tc_full_body,
        grid=(N_BLOCKS,),
        in_specs=[
            pl.BlockSpec((BLOCK, DIM), lambda i: (i, 0)),
            pl.BlockSpec((DIM, N_EMB), lambda i: (0, 0)),
            pl.BlockSpec((N_EMB, DIM), lambda i: (0, 0)),
        ],
        out_specs=[
            pl.BlockSpec((BLOCK, DIM), lambda i: (i, 0)),
            pl.BlockSpec((BLOCK, 1), lambda i: (i, 0)),
            pl.BlockSpec((1, 1), lambda i: (0, 0), memory_space=pltpu.SMEM),
        ],
        out_shape=[
            jax.ShapeDtypeStruct((ROWS, DIM), jnp.float32),
            jax.ShapeDtypeStruct((ROWS, 1), jnp.int32),
            jax.ShapeDtypeStruct((1, 1), jnp.float32),
        ],
    )(flat, wt, w)


def kernel(inputs, weight):
    flat = inputs.reshape(ROWS, DIM)
    idx, loss = _tc_call(flat, weight.T)
    quantized = _sc_gather()(weight.reshape(N_EMB // 2, 2 * DIM), idx)
    return (quantized.reshape(inputs.shape), loss[0, 0], idx)


# speedup vs baseline: 1.3258x; 1.2047x over previous
import functools

import jax
import jax.numpy as jnp
from jax import lax
from jax.experimental import pallas as pl
from jax.experimental.pallas import tpu as pltpu
from jax.experimental.pallas import tpu_sc as plsc

N_EMB = 1024
DIM = 64
ROWS = 8 * 576
BLOCK = 576
N_BLOCKS = ROWS // BLOCK

N_WORKERS = 32
_B_PER_W = ROWS // N_WORKERS
_CH = _B_PER_W // 2
LOSS_SCALE = 0.5 / (ROWS * DIM)


def _tc_body(x_ref, wt_ref, idx_ref, loss_ref):
    x = x_ref[...]
    wt = wt_ref[...]

    num = jnp.dot(x, wt, preferred_element_type=jnp.float32)
    xsq = jnp.sum(x * x, axis=1, keepdims=True)
    wnsq = jnp.sum(wt * wt, axis=0, keepdims=True)
    x_norm = jnp.sqrt(xsq)
    w_norm = jnp.sqrt(wnsq)
    denom = jnp.maximum(x_norm * w_norm, 1e-8)
    dist = 1.0 - num / denom

    m = jnp.min(dist, axis=1, keepdims=True)
    iota = lax.broadcasted_iota(jnp.int32, (BLOCK, N_EMB), 1)
    idx = jnp.min(jnp.where(dist == m, iota, N_EMB), axis=1, keepdims=True)
    idx_ref[...] = idx

    sel = iota == idx
    num_sel = jnp.sum(jnp.where(sel, num, 0.0), axis=1)
    wnsq_sel = jnp.sum(jnp.where(sel, wnsq, 0.0), axis=1)
    block_loss = jnp.sum(xsq[:, 0] - 2.0 * num_sel + wnsq_sel)

    i = pl.program_id(0)

    @pl.when(i == 0)
    def _():
        loss_ref[0, 0] = 0.0

    loss_ref[0, 0] += block_loss * LOSS_SCALE


def _tc_call(flat, wt):
    return pl.pallas_call(
        _tc_body,
        grid=(N_BLOCKS,),
        in_specs=[
            pl.BlockSpec((BLOCK, DIM), lambda i: (i, 0)),
            pl.BlockSpec((DIM, N_EMB), lambda i: (0, 0)),
        ],
        out_specs=[
            pl.BlockSpec((BLOCK, 1), lambda i: (i, 0)),
            pl.BlockSpec((1, 1), lambda i: (0, 0), memory_space=pltpu.SMEM),
        ],
        out_shape=[
            jax.ShapeDtypeStruct((ROWS, 1), jnp.int32),
            jax.ShapeDtypeStruct((1, 1), jnp.float32),
        ],
    )(flat, wt)


def _sc_body(table_hbm, idx_hbm, out_hbm, idx_v, rows_v, sem):
    wid = lax.axis_index("s") * 2 + lax.axis_index("c")
    base = wid * _B_PER_W
    pltpu.sync_copy(idx_hbm.at[pl.ds(wid * 2, 2)], idx_v)
    cp0 = pltpu.async_copy(table_hbm.at[idx_v.at[0]],
                           rows_v.at[pl.ds(0, _CH)], sem)
    cp1 = pltpu.async_copy(table_hbm.at[idx_v.at[1]],
                           rows_v.at[pl.ds(_CH, _CH)], sem)
    cp0.wait()
    cp1.wait()
    pltpu.sync_copy(rows_v, out_hbm.at[pl.ds(base, _B_PER_W)])


@functools.cache
def _sc_gather():
    return pl.kernel(
        _sc_body,
        out_type=jax.ShapeDtypeStruct((ROWS, DIM), jnp.float32),
        mesh=plsc.VectorSubcoreMesh(core_axis_name="c", subcore_axis_name="s"),
        scratch_types=[
            pltpu.VMEM((2, _CH), jnp.int32),
            pltpu.VMEM((_B_PER_W, DIM), jnp.float32),
            pltpu.SemaphoreType.DMA,
        ],
        compiler_params=pltpu.CompilerParams(use_tc_tiling_on_sc=False,
                                             skip_device_barrier=True),
    )


def _tc_full_body(x_ref, wt_ref, w_ref, q_ref, idx_ref, loss_ref):
    i = pl.program_id(0)
    x = x_ref[...]
    wt = wt_ref[...]

    num = jnp.dot(x, wt, preferred_element_type=jnp.float32)
    x_norm = jnp.sqrt(jnp.sum(x * x, axis=1, keepdims=True))
    w_norm = jnp.sqrt(jnp.sum(wt * wt, axis=0, keepdims=True))
    denom = jnp.maximum(x_norm * w_norm, 1e-8)
    dist = 1.0 - num / denom

    m = jnp.min(dist, axis=1, keepdims=True)
    iota = lax.broadcasted_iota(jnp.int32, (BLOCK, N_EMB), 1)
    idx = jnp.min(jnp.where(dist == m, iota, N_EMB), axis=1, keepdims=True)
    idx_ref[...] = idx

    onehot = (iota == idx).astype(jnp.float32)
    q = jnp.dot(onehot, w_ref[...], preferred_element_type=jnp.float32)
    q_ref[...] = q

    d = q - x
    block_loss = jnp.sum(d * d)

    @pl.when(i == 0)
    def _():
        loss_ref[0, 0] = 0.0

    loss_ref[0, 0] += block_loss * LOSS_SCALE


def _tc_full_call(flat, wt, w):
    return pl.pallas_call(
        _tc_full_body,
        grid=(N_BLOCKS,),
        in_specs=[
            pl.BlockSpec((BLOCK, DIM), lambda i: (i, 0)),
            pl.BlockSpec((DIM, N_EMB), lambda i: (0, 0)),
            pl.BlockSpec((N_EMB, DIM), lambda i: (0, 0)),
        ],
        out_specs=[
            pl.BlockSpec((BLOCK, DIM), lambda i: (i, 0)),
            pl.BlockSpec((BLOCK, 1), lambda i: (i, 0)),
            pl.BlockSpec((1, 1), lambda i: (0, 0), memory_space=pltpu.SMEM),
        ],
        out_shape=[
            jax.ShapeDtypeStruct((ROWS, DIM), jnp.float32),
            jax.ShapeDtypeStruct((ROWS, 1), jnp.int32),
            jax.ShapeDtypeStruct((1, 1), jnp.float32),
        ],
    )(flat, wt, w)


def kernel(inputs, weight):
    flat = inputs.reshape(ROWS, DIM)
    idx, loss = _tc_call(flat, weight.T)
    quantized = _sc_gather()(weight, idx.reshape(ROWS // _CH, _CH))
    return (quantized.reshape(inputs.shape), loss[0, 0], idx)


# speedup vs baseline: 1.3724x; 1.0351x over previous
import functools

import jax
import jax.numpy as jnp
from jax import lax
from jax.experimental import pallas as pl
from jax.experimental.pallas import tpu as pltpu
from jax.experimental.pallas import tpu_sc as plsc

N_EMB = 1024
DIM = 64
ROWS = 8 * 576
BLOCK = 512
N_BLOCKS = ROWS // BLOCK

N_WORKERS = 32
_B_PER_W = ROWS // N_WORKERS
_CH = _B_PER_W // 2
LOSS_SCALE = 0.5 / (ROWS * DIM)


def _tc_body(x_ref, wt_ref, idx_ref, idx2_ref, loss_ref):
    x = x_ref[...]
    wt = wt_ref[...]

    num = jnp.dot(x, wt, preferred_element_type=jnp.float32)
    xsq = jnp.sum(x * x, axis=1, keepdims=True)
    wnsq = jnp.sum(wt * wt, axis=0, keepdims=True)
    x_norm = jnp.sqrt(xsq)
    w_norm = jnp.sqrt(wnsq)
    denom = jnp.maximum(x_norm * w_norm, 1e-8)
    dist = 1.0 - num / denom

    m = jnp.min(dist, axis=1, keepdims=True)
    iota = lax.broadcasted_iota(jnp.int32, (BLOCK, N_EMB), 1)
    idx = jnp.min(jnp.where(dist == m, iota, N_EMB), axis=1, keepdims=True)
    idx_ref[...] = idx
    idx2_ref[...] = idx.reshape(1, BLOCK // 128, 128)

    sel = iota == idx
    num_sel = jnp.sum(jnp.where(sel, num, 0.0), axis=1)
    wnsq_sel = jnp.sum(jnp.where(sel, wnsq, 0.0), axis=1)
    block_loss = jnp.sum(xsq[:, 0] - 2.0 * num_sel + wnsq_sel)

    i = pl.program_id(0)

    @pl.when(i == 0)
    def _():
        loss_ref[0, 0] = 0.0

    loss_ref[0, 0] += block_loss * LOSS_SCALE


def _tc_call(flat, wt):
    return pl.pallas_call(
        _tc_body,
        grid=(N_BLOCKS,),
        in_specs=[
            pl.BlockSpec((BLOCK, DIM), lambda i: (i, 0)),
            pl.BlockSpec((DIM, N_EMB), lambda i: (0, 0)),
        ],
        out_specs=[
            pl.BlockSpec((BLOCK, 1), lambda i: (i, 0)),
            pl.BlockSpec((1, BLOCK // 128, 128), lambda i: (i, 0, 0)),
            pl.BlockSpec((1, 1), lambda i: (0, 0), memory_space=pltpu.SMEM),
        ],
        out_shape=[
            jax.ShapeDtypeStruct((ROWS, 1), jnp.int32),
            jax.ShapeDtypeStruct((N_BLOCKS, BLOCK // 128, 128), jnp.int32),
            jax.ShapeDtypeStruct((1, 1), jnp.float32),
        ],
    )(flat, wt)


def _sc_body(table_hbm, idx_hbm, out_hbm, idx_v, rows_v, sem):
    wid = lax.axis_index("s") * 2 + lax.axis_index("c")
    base = wid * _B_PER_W
    pltpu.sync_copy(idx_hbm.at[pl.ds(wid * 2, 2)], idx_v)
    cp0 = pltpu.async_copy(table_hbm.at[idx_v.at[0]],
                           rows_v.at[pl.ds(0, _CH)], sem)
    cp1 = pltpu.async_copy(table_hbm.at[idx_v.at[1]],
                           rows_v.at[pl.ds(_CH, _CH)], sem)
    cp0.wait()
    cp1.wait()
    pltpu.sync_copy(rows_v, out_hbm.at[pl.ds(base, _B_PER_W)])


@functools.cache
def _sc_gather():
    return pl.kernel(
        _sc_body,
        out_type=jax.ShapeDtypeStruct((ROWS, DIM), jnp.float32),
        mesh=plsc.VectorSubcoreMesh(core_axis_name="c", subcore_axis_name="s"),
        scratch_types=[
            pltpu.VMEM((2, _CH), jnp.int32),
            pltpu.VMEM((_B_PER_W, DIM), jnp.float32),
            pltpu.SemaphoreType.DMA,
        ],
        compiler_params=pltpu.CompilerParams(use_tc_tiling_on_sc=False,
                                             skip_device_barrier=True),
    )


def kernel(inputs, weight):
    flat = inputs.reshape(ROWS, DIM)
    idx, idx2, loss = _tc_call(flat, weight.T)
    quantized = _sc_gather()(weight, idx2.reshape(ROWS // _CH, _CH))
    return (quantized.reshape(inputs.shape), loss[0, 0], idx)


# speedup vs baseline: 1.4064x; 1.0248x over previous
import functools

import jax
import jax.numpy as jnp
from jax import lax
from jax.experimental import pallas as pl
from jax.experimental.pallas import tpu as pltpu
from jax.experimental.pallas import tpu_sc as plsc

N_EMB = 1024
DIM = 64
ROWS = 8 * 576
BLOCK = 512
N_BLOCKS = ROWS // BLOCK

N_WORKERS = 32
_B_PER_W = ROWS // N_WORKERS
_CH = _B_PER_W // 2
LOSS_SCALE = 0.5 / (ROWS * DIM)


def _tc_body(x_ref, wt_ref, idx_ref, idx2_ref, loss_ref):
    x = x_ref[...]
    wt = wt_ref[...]

    num = jnp.dot(x, wt, preferred_element_type=jnp.float32)
    xsq = jnp.sum(x * x, axis=1, keepdims=True)
    wnsq = jnp.sum(wt * wt, axis=0, keepdims=True)
    x_norm = jnp.sqrt(xsq)
    w_norm = jnp.sqrt(wnsq)
    denom = jnp.maximum(x_norm * w_norm, 1e-8)
    dist = 1.0 - num / denom

    m = jnp.min(dist, axis=1, keepdims=True)
    iota = lax.broadcasted_iota(jnp.int32, (BLOCK, N_EMB), 1)
    idx = jnp.min(jnp.where(dist == m, iota, N_EMB), axis=1, keepdims=True)
    idx_ref[...] = idx
    idx2_ref[...] = idx.reshape(1, BLOCK // 128, 128)

    sel = iota == idx
    wn_sel = jnp.sum(jnp.where(sel, w_norm, 0.0), axis=1)
    denom_sel = jnp.maximum(x_norm[:, 0] * wn_sel, 1e-8)
    num_sel = (1.0 - m[:, 0]) * denom_sel
    block_loss = jnp.sum(xsq[:, 0] - 2.0 * num_sel + wn_sel * wn_sel)

    i = pl.program_id(0)

    @pl.when(i == 0)
    def _():
        loss_ref[0, 0] = 0.0

    loss_ref[0, 0] += block_loss * LOSS_SCALE


def _tc_call(flat, wt):
    return pl.pallas_call(
        _tc_body,
        grid=(N_BLOCKS,),
        in_specs=[
            pl.BlockSpec((BLOCK, DIM), lambda i: (i, 0)),
            pl.BlockSpec((DIM, N_EMB), lambda i: (0, 0)),
        ],
        out_specs=[
            pl.BlockSpec((BLOCK, 1), lambda i: (i, 0)),
            pl.BlockSpec((1, BLOCK // 128, 128), lambda i: (i, 0, 0)),
            pl.BlockSpec((1, 1), lambda i: (0, 0), memory_space=pltpu.SMEM),
        ],
        out_shape=[
            jax.ShapeDtypeStruct((ROWS, 1), jnp.int32),
            jax.ShapeDtypeStruct((N_BLOCKS, BLOCK // 128, 128), jnp.int32),
            jax.ShapeDtypeStruct((1, 1), jnp.float32),
        ],
    )(flat, wt)


def _sc_body(table_hbm, idx_hbm, out_hbm, idx_v, rows_v, sem):
    wid = lax.axis_index("s") * 2 + lax.axis_index("c")
    base = wid * _B_PER_W
    pltpu.sync_copy(idx_hbm.at[pl.ds(wid * 2, 2)], idx_v)
    cp0 = pltpu.async_copy(table_hbm.at[idx_v.at[0]],
                           rows_v.at[pl.ds(0, _CH)], sem)
    cp1 = pltpu.async_copy(table_hbm.at[idx_v.at[1]],
                           rows_v.at[pl.ds(_CH, _CH)], sem)
    cp0.wait()
    cp1.wait()
    pltpu.sync_copy(rows_v, out_hbm.at[pl.ds(base, _B_PER_W)])


@functools.cache
def _sc_gather():
    return pl.kernel(
        _sc_body,
        out_type=jax.ShapeDtypeStruct((ROWS, DIM), jnp.float32),
        mesh=plsc.VectorSubcoreMesh(core_axis_name="c", subcore_axis_name="s"),
        scratch_types=[
            pltpu.VMEM((2, _CH), jnp.int32),
            pltpu.VMEM((_B_PER_W, DIM), jnp.float32),
            pltpu.SemaphoreType.DMA,
        ],
        compiler_params=pltpu.CompilerParams(use_tc_tiling_on_sc=False,
                                             skip_device_barrier=True),
    )


def kernel(inputs, weight):
    flat = inputs.reshape(ROWS, DIM)
    idx, idx2, loss = _tc_call(flat, weight.T)
    quantized = _sc_gather()(weight, idx2.reshape(ROWS // _CH, _CH))
    return (quantized.reshape(inputs.shape), loss[0, 0], idx)


# speedup vs baseline: 1.4163x; 1.0070x over previous
import functools

import jax
import jax.numpy as jnp
from jax import lax
from jax.experimental import pallas as pl
from jax.experimental.pallas import tpu as pltpu
from jax.experimental.pallas import tpu_sc as plsc

N_EMB = 1024
DIM = 64
ROWS = 8 * 576
BLOCK = 512
N_BLOCKS = ROWS // BLOCK

N_WORKERS = 32
_B_PER_W = ROWS // N_WORKERS
_CH = _B_PER_W // 2
LOSS_SCALE = 0.5 / (ROWS * DIM)


def _tc_body(x_ref, wt_ref, idx2_ref, loss_ref):
    x = x_ref[...]
    wt = wt_ref[...]

    num = jnp.dot(x, wt, preferred_element_type=jnp.float32)
    xsq = jnp.sum(x * x, axis=1, keepdims=True)
    wnsq = jnp.sum(wt * wt, axis=0, keepdims=True)
    x_norm = jnp.sqrt(xsq)
    w_norm = jnp.sqrt(wnsq)
    denom = jnp.maximum(x_norm * w_norm, 1e-8)
    dist = 1.0 - num / denom

    m = jnp.min(dist, axis=1, keepdims=True)
    iota = lax.broadcasted_iota(jnp.int32, (BLOCK, N_EMB), 1)
    idx = jnp.min(jnp.where(dist == m, iota, N_EMB), axis=1, keepdims=True)
    idx2_ref[...] = idx.reshape(1, BLOCK // 128, 128)

    sel = iota == idx
    wn_sel = jnp.sum(jnp.where(sel, w_norm, 0.0), axis=1)
    denom_sel = jnp.maximum(x_norm[:, 0] * wn_sel, 1e-8)
    num_sel = (1.0 - m[:, 0]) * denom_sel
    block_loss = jnp.sum(xsq[:, 0] - 2.0 * num_sel + wn_sel * wn_sel)

    i = pl.program_id(0)

    @pl.when(i == 0)
    def _():
        loss_ref[0, 0] = 0.0

    loss_ref[0, 0] += block_loss * LOSS_SCALE


def _tc_call(flat, wt):
    return pl.pallas_call(
        _tc_body,
        grid=(N_BLOCKS,),
        in_specs=[
            pl.BlockSpec((BLOCK, DIM), lambda i: (i, 0)),
            pl.BlockSpec((DIM, N_EMB), lambda i: (0, 0)),
        ],
        out_specs=[
            pl.BlockSpec((1, BLOCK // 128, 128), lambda i: (i, 0, 0)),
            pl.BlockSpec((1, 1), lambda i: (0, 0), memory_space=pltpu.SMEM),
        ],
        out_shape=[
            jax.ShapeDtypeStruct((N_BLOCKS, BLOCK // 128, 128), jnp.int32),
            jax.ShapeDtypeStruct((1, 1), jnp.float32),
        ],
    )(flat, wt)


def _sc_body(table_hbm, idx_hbm, out_hbm, idx_v, rows_v, sem):
    wid = lax.axis_index("s") * 2 + lax.axis_index("c")
    base = wid * _B_PER_W
    pltpu.sync_copy(idx_hbm.at[pl.ds(wid * 2, 2)], idx_v)
    cp0 = pltpu.async_copy(table_hbm.at[idx_v.at[0]],
                           rows_v.at[pl.ds(0, _CH)], sem)
    cp1 = pltpu.async_copy(table_hbm.at[idx_v.at[1]],
                           rows_v.at[pl.ds(_CH, _CH)], sem)
    cp0.wait()
    cp1.wait()
    pltpu.sync_copy(rows_v, out_hbm.at[pl.ds(base, _B_PER_W)])


@functools.cache
def _sc_gather():
    return pl.kernel(
        _sc_body,
        out_type=jax.ShapeDtypeStruct((ROWS, DIM), jnp.float32),
        mesh=plsc.VectorSubcoreMesh(core_axis_name="c", subcore_axis_name="s"),
        scratch_types=[
            pltpu.VMEM((2, _CH), jnp.int32),
            pltpu.VMEM((_B_PER_W, DIM), jnp.float32),
            pltpu.SemaphoreType.DMA,
        ],
        compiler_params=pltpu.CompilerParams(use_tc_tiling_on_sc=False,
                                             skip_device_barrier=True),
    )


def kernel(inputs, weight):
    flat = inputs.reshape(ROWS, DIM)
    idx2, loss = _tc_call(flat, weight.T)
    quantized = _sc_gather()(weight, idx2.reshape(ROWS // _CH, _CH))
    return (quantized.reshape(inputs.shape), loss[0, 0],
            idx2.reshape(ROWS, 1))


# speedup vs baseline: 1.4419x; 1.0181x over previous
import functools

import jax
import jax.numpy as jnp
from jax import lax
from jax.experimental import pallas as pl
from jax.experimental.pallas import tpu as pltpu
from jax.experimental.pallas import tpu_sc as plsc

N_EMB = 1024
DIM = 64
ROWS = 8 * 576
BLOCK = 1152
N_BLOCKS = ROWS // BLOCK

N_WORKERS = 32
_B_PER_W = ROWS // N_WORKERS
_CH = _B_PER_W // 2
LOSS_SCALE = 0.5 / (ROWS * DIM)


def _tc_body(x_ref, wt_ref, idx2_ref, loss_ref):
    x = x_ref[...]
    wt = wt_ref[...]

    num = jnp.dot(x, wt, preferred_element_type=jnp.float32)
    xsq = jnp.sum(x * x, axis=1, keepdims=True)
    wnsq = jnp.sum(wt * wt, axis=0, keepdims=True)
    x_norm = jnp.sqrt(xsq)
    w_norm = jnp.sqrt(wnsq)
    denom = jnp.maximum(x_norm * w_norm, 1e-8)
    dist = 1.0 - num / denom

    m = jnp.min(dist, axis=1, keepdims=True)
    iota = lax.broadcasted_iota(jnp.int32, (BLOCK, N_EMB), 1)
    idx = jnp.min(jnp.where(dist == m, iota, N_EMB), axis=1, keepdims=True)
    idx2_ref[...] = idx.reshape(1, BLOCK // 128, 128)

    sel = iota == idx
    wn_sel = jnp.sum(jnp.where(sel, w_norm, 0.0), axis=1)
    denom_sel = jnp.maximum(x_norm[:, 0] * wn_sel, 1e-8)
    num_sel = (1.0 - m[:, 0]) * denom_sel
    block_loss = jnp.sum(xsq[:, 0] - 2.0 * num_sel + wn_sel * wn_sel)

    i = pl.program_id(0)

    @pl.when(i == 0)
    def _():
        loss_ref[0, 0] = 0.0

    loss_ref[0, 0] += block_loss * LOSS_SCALE


def _tc_call(flat, wt):
    return pl.pallas_call(
        _tc_body,
        grid=(N_BLOCKS,),
        in_specs=[
            pl.BlockSpec((BLOCK, DIM), lambda i: (i, 0)),
            pl.BlockSpec((DIM, N_EMB), lambda i: (0, 0)),
        ],
        out_specs=[
            pl.BlockSpec((1, BLOCK // 128, 128), lambda i: (i, 0, 0)),
            pl.BlockSpec((1, 1), lambda i: (0, 0), memory_space=pltpu.SMEM),
        ],
        out_shape=[
            jax.ShapeDtypeStruct((N_BLOCKS, BLOCK // 128, 128), jnp.int32),
            jax.ShapeDtypeStruct((1, 1), jnp.float32),
        ],
    )(flat, wt)


def _sc_body(table_hbm, idx_hbm, out_hbm, idx_v, rows_v, sem):
    wid = lax.axis_index("s") * 2 + lax.axis_index("c")
    base = wid * _B_PER_W
    pltpu.sync_copy(idx_hbm.at[pl.ds(wid * 2, 2)], idx_v)
    cp0 = pltpu.async_copy(table_hbm.at[idx_v.at[0]],
                           rows_v.at[pl.ds(0, _CH)], sem)
    cp1 = pltpu.async_copy(table_hbm.at[idx_v.at[1]],
                           rows_v.at[pl.ds(_CH, _CH)], sem)
    cp0.wait()
    cp1.wait()
    pltpu.sync_copy(rows_v, out_hbm.at[pl.ds(base, _B_PER_W)])


@functools.cache
def _sc_gather():
    return pl.kernel(
        _sc_body,
        out_type=jax.ShapeDtypeStruct((ROWS, DIM), jnp.float32),
        mesh=plsc.VectorSubcoreMesh(core_axis_name="c", subcore_axis_name="s"),
        scratch_types=[
            pltpu.VMEM((2, _CH), jnp.int32),
            pltpu.VMEM((_B_PER_W, DIM), jnp.float32),
            pltpu.SemaphoreType.DMA,
        ],
        compiler_params=pltpu.CompilerParams(use_tc_tiling_on_sc=False,
                                             skip_device_barrier=True),
    )


def kernel(inputs, weight):
    flat = inputs.reshape(ROWS, DIM)
    idx2, loss = _tc_call(flat, weight.T)
    quantized = _sc_gather()(weight, idx2.reshape(ROWS // _CH, _CH))
    return (quantized.reshape(inputs.shape), loss[0, 0],
            idx2.reshape(ROWS, 1))


# speedup vs baseline: 1.4458x; 1.0027x over previous
import functools

import jax
import jax.numpy as jnp
from jax import lax
from jax.experimental import pallas as pl
from jax.experimental.pallas import tpu as pltpu
from jax.experimental.pallas import tpu_sc as plsc

N_EMB = 1024
DIM = 64
ROWS = 8 * 576
BLOCK = 1152
N_BLOCKS = ROWS // BLOCK

N_WORKERS = 32
_B_PER_W = ROWS // N_WORKERS
_CH = _B_PER_W // 2
LOSS_SCALE = 0.5 / (ROWS * DIM)


def _tc_body(x_ref, wt_ref, idx2_ref, loss_ref):
    x = x_ref[...]
    wt = wt_ref[...]

    num = jnp.dot(x, wt, preferred_element_type=jnp.float32)
    xsq = jnp.sum(x * x, axis=1, keepdims=True)
    wnsq = jnp.sum(wt * wt, axis=0, keepdims=True)
    x_norm = jnp.sqrt(xsq)
    w_norm = jnp.sqrt(wnsq)
    denom = jnp.maximum(x_norm * w_norm, 1e-8)
    dist = 1.0 - num / denom

    m = jnp.min(dist, axis=1, keepdims=True)
    iota = lax.broadcasted_iota(jnp.int32, (BLOCK, N_EMB), 1)
    idx = jnp.min(jnp.where(dist == m, iota, N_EMB), axis=1, keepdims=True)
    idx2_ref[...] = idx.reshape(1, BLOCK // 128, 128)

    sel = iota == idx
    wn_sel = jnp.sum(jnp.where(sel, w_norm, 0.0), axis=1)
    denom_sel = jnp.maximum(x_norm[:, 0] * wn_sel, 1e-8)
    num_sel = (1.0 - m[:, 0]) * denom_sel
    block_loss = jnp.sum(xsq[:, 0] - 2.0 * num_sel + wn_sel * wn_sel)

    i = pl.program_id(0)

    @pl.when(i == 0)
    def _():
        loss_ref[0, 0] = 0.0

    loss_ref[0, 0] += block_loss * LOSS_SCALE


def _tc_call(flat, wt):
    return pl.pallas_call(
        _tc_body,
        grid=(N_BLOCKS,),
        in_specs=[
            pl.BlockSpec((BLOCK, DIM), lambda i: (i, 0)),
            pl.BlockSpec((DIM, N_EMB), lambda i: (0, 0)),
        ],
        out_specs=[
            pl.BlockSpec((1, BLOCK // 128, 128), lambda i: (i, 0, 0)),
            pl.BlockSpec((1, 1), lambda i: (0, 0), memory_space=pltpu.SMEM),
        ],
        out_shape=[
            jax.ShapeDtypeStruct((N_BLOCKS, BLOCK // 128, 128), jnp.int32),
            jax.ShapeDtypeStruct((1, 1), jnp.float32),
        ],
    )(flat, wt)


def _sc_body(table_hbm, idx_hbm, out_hbm, idx_v, rows_v, sem):
    wid = lax.axis_index("s") * 2 + lax.axis_index("c")
    base = wid * _B_PER_W
    pltpu.sync_copy(idx_hbm.at[pl.ds(wid * 2, 2)], idx_v)
    cp0 = pltpu.async_copy(table_hbm.at[idx_v.at[0]],
                           rows_v.at[pl.ds(0, _CH)], sem)
    cp1 = pltpu.async_copy(table_hbm.at[idx_v.at[1]],
                           rows_v.at[pl.ds(_CH, _CH)], sem)
    cp0.wait()
    cp1.wait()
    pltpu.sync_copy(rows_v, out_hbm.at[pl.ds(base, _B_PER_W)])


@functools.cache
def _sc_gather():
    return pl.kernel(
        _sc_body,
        out_type=jax.ShapeDtypeStruct((ROWS, DIM), jnp.float32),
        mesh=plsc.VectorSubcoreMesh(core_axis_name="c", subcore_axis_name="s"),
        scratch_types=[
            pltpu.VMEM((2, _CH), jnp.int32),
            pltpu.VMEM((_B_PER_W, DIM), jnp.float32),
            pltpu.SemaphoreType.DMA,
        ],
        compiler_params=pltpu.CompilerParams(use_tc_tiling_on_sc=False),
    )


def kernel(inputs, weight):
    flat = inputs.reshape(ROWS, DIM)
    idx2, loss = _tc_call(flat, weight.T)
    quantized = _sc_gather()(weight, idx2.reshape(ROWS // _CH, _CH))
    return (quantized.reshape(inputs.shape), loss[0, 0],
            idx2.reshape(ROWS, 1))
